# Initial kernel scaffold; baseline (speedup 1.0000x reference)
import jax
import jax.numpy as jnp
from jax.experimental import pallas as pl


def kernel(x, edge_index, W, b):
    raise NotImplementedError("write your pallas kernel here")


# speedup vs baseline: 8.6178x; 8.6178x over previous
import functools

import jax
import jax.numpy as jnp
from jax import lax
from jax.experimental import pallas as pl
from jax.experimental.pallas import tpu as pltpu
from jax.experimental.pallas import tpu_sc as plsc

_NC = 2
_NS = 16
_NW = _NC * _NS
_CH = 128


def _build_deg(n_chunks_w, n_pad):
    mesh = plsc.VectorSubcoreMesh(core_axis_name="c", subcore_axis_name="s")
    rpt = n_pad // _NS
    zr = rpt // 8

    @functools.partial(
        pl.kernel,
        mesh=mesh,
        out_type=jax.ShapeDtypeStruct((_NC, n_pad, 16), jnp.float32),
        scratch_types=[
            pltpu.VMEM((n_chunks_w, _CH), jnp.int32),
            pltpu.VMEM((_CH, 16), jnp.float32),
            pltpu.VMEM((zr, 16), jnp.float32),
            pltpu.VMEM_SHARED((n_pad + 8, 16), jnp.float32),
        ],
    )
    def degk(dst_hbm, out_hbm, didx, ones_v, zv, hist):
        c = lax.axis_index("c")
        s = lax.axis_index("s")
        w = c * _NS + s

        def fill_ones(i, _):
            ones_v[i, :] = jnp.ones((16,), jnp.float32)
            return 0

        lax.fori_loop(0, _CH, fill_ones, 0)

        def fill_zero(i, _):
            zv[i, :] = jnp.zeros((16,), jnp.float32)
            return 0

        lax.fori_loop(0, zr, fill_zero, 0)

        r0 = s * rpt
        for j in range(8):
            pltpu.sync_copy(zv, hist.at[pl.ds(r0 + j * zr, zr)])
        plsc.subcore_barrier()

        pltpu.sync_copy(dst_hbm.at[pl.ds(w * n_chunks_w, n_chunks_w)], didx)

        def body(i, _):
            pltpu.sync_copy(ones_v, hist.at[didx.at[i]], add=True)
            return 0

        lax.fori_loop(0, n_chunks_w, body, 0)
        plsc.subcore_barrier()
        pltpu.sync_copy(hist.at[pl.ds(r0, rpt)], out_hbm.at[c, pl.ds(r0, rpt)])

    return degk


def _build_prop(n_chunks_w, n_pad, d):
    mesh = plsc.VectorSubcoreMesh(core_axis_name="c", subcore_axis_name="s")
    rpt = n_pad // _NS
    zr = rpt // 8

    @functools.partial(
        pl.kernel,
        mesh=mesh,
        out_type=jax.ShapeDtypeStruct((_NC, n_pad, d), jnp.float32),
        scratch_types=[
            pltpu.VMEM((n_chunks_w, _CH), jnp.int32),
            pltpu.VMEM((n_chunks_w, _CH), jnp.int32),
            pltpu.VMEM((_CH, d), jnp.float32),
            pltpu.VMEM((zr, d), jnp.float32),
            pltpu.VMEM_SHARED((n_pad + 8, d), jnp.float32),
            pltpu.SemaphoreType.DMA,
        ],
    )
    def prop(src_hbm, dst_hbm, y_hbm, out_hbm, sidx, didx, rows, zv, acc, sem):
        c = lax.axis_index("c")
        s = lax.axis_index("s")
        w = c * _NS + s

        def fill_zero(i, _):
            r = i // (d // 16)
            col = (i % (d // 16)) * 16
            zv[r, pl.ds(col, 16)] = jnp.zeros((16,), jnp.float32)
            return 0

        lax.fori_loop(0, zr * (d // 16), fill_zero, 0)

        r0 = s * rpt
        for j in range(8):
            pltpu.sync_copy(zv, acc.at[pl.ds(r0 + j * zr, zr)])
        plsc.subcore_barrier()

        cr0 = w * n_chunks_w
        pltpu.sync_copy(src_hbm.at[pl.ds(cr0, n_chunks_w)], sidx)
        pltpu.sync_copy(dst_hbm.at[pl.ds(cr0, n_chunks_w)], didx)

        def body(i, _):
            pltpu.async_copy(y_hbm.at[sidx.at[i]], rows, sem).wait()
            pltpu.sync_copy(rows, acc.at[didx.at[i]], add=True)
            return 0

        lax.fori_loop(0, n_chunks_w, body, 0)
        plsc.subcore_barrier()
        pltpu.sync_copy(acc.at[pl.ds(r0, rpt)], out_hbm.at[c, pl.ds(r0, rpt)])

    return prop


def _deg_from_cnt(cnt_ref):
    return 1.0 + cnt_ref[0, :, 0:1] + cnt_ref[1, :, 0:1]


def _tc1_body(cnt_ref, x_ref, y_ref):
    y_ref[...] = x_ref[...] * lax.rsqrt(_deg_from_cnt(cnt_ref))


def _tc2_body(cnt_ref, y0_ref, p_ref, o_ref):
    o_ref[...] = (y0_ref[...] + p_ref[0] + p_ref[1]) / _deg_from_cnt(cnt_ref)


def _tc3_body(cnt_ref, y1_ref, q_ref, w_ref, b_ref, o_ref):
    h2 = (y1_ref[...] + q_ref[0] + q_ref[1]) * lax.rsqrt(_deg_from_cnt(cnt_ref))
    o = lax.dot_general(
        h2, w_ref[...], (((1,), (0,)), ((), ())),
        precision=lax.Precision.HIGHEST,
        preferred_element_type=jnp.float32,
    ) + b_ref[...]
    m = jnp.max(o, axis=1, keepdims=True)
    lse = m + jnp.log(jnp.sum(jnp.exp(o - m), axis=1, keepdims=True))
    o_ref[...] = o - lse


def kernel(x, edge_index, W, b):
    n, d = x.shape
    e = edge_index.shape[1]

    n_chunks_w = -(-e // (_NW * _CH * 8)) * 8
    ep = _NW * _CH * n_chunks_w
    pad = ep - e
    src = jnp.concatenate(
        [edge_index[0], jnp.zeros((pad,), edge_index.dtype)])
    dst = jnp.concatenate(
        [edge_index[1], jnp.full((pad,), n, edge_index.dtype)])
    src2 = src.reshape(ep // _CH, _CH)
    dst2 = dst.reshape(ep // _CH, _CH)

    n_pad = -(-n // (_NS * 8)) * (_NS * 8)

    degk = _build_deg(n_chunks_w, n_pad)
    prop = _build_prop(n_chunks_w, n_pad, d)

    cnt = degk(dst2)

    br = 1000
    grid = (n // br,)
    cnt_spec = pl.BlockSpec((2, br, 16), lambda i: (0, i, 0))
    row_spec = pl.BlockSpec((br, d), lambda i: (i, 0))
    par_spec = pl.BlockSpec((2, br, d), lambda i: (0, i, 0))
    out_sds = jax.ShapeDtypeStruct((n, d), jnp.float32)

    y0 = pl.pallas_call(
        _tc1_body,
        grid=grid,
        in_specs=[cnt_spec, row_spec],
        out_specs=row_spec,
        out_shape=out_sds,
    )(cnt, x)

    p = prop(src2, dst2, y0)

    y1 = pl.pallas_call(
        _tc2_body,
        grid=grid,
        in_specs=[cnt_spec, row_spec, par_spec],
        out_specs=row_spec,
        out_shape=out_sds,
    )(cnt, y0, p)

    q = prop(src2, dst2, y1)

    out = pl.pallas_call(
        _tc3_body,
        grid=grid,
        in_specs=[
            cnt_spec,
            row_spec,
            par_spec,
            pl.BlockSpec((d, d), lambda i: (0, 0)),
            pl.BlockSpec((1, d), lambda i: (0, 0)),
        ],
        out_specs=row_spec,
        out_shape=out_sds,
    )(cnt, y1, q, W, b.reshape(1, d))

    return out
